# baseline (speedup 1.0000x reference)
import functools

import jax
import jax.numpy as jnp
from jax import lax
from jax.experimental import pallas as pl
from jax.experimental.pallas import tpu as pltpu
from jax.experimental.pallas import tpu_sc as plsc

NC = 2
NS = 16
LN = 16
EB = 128
CH = 200


def _sc_chebyshev(M, F, NB, xin, cv, rows):
    nchk = M // CH
    nit = -(-nchk // NS)

    dnums = lax.GatherDimensionNumbers(
        offset_dims=(), collapsed_slice_dims=(0,), start_index_map=(0,))

    def body(xin_hbm, cv_hbm, r_hbm, x1_hbm, x2_hbm, x3_hbm,
             cvb, rbuf, gbuf, acc_sh,
             sem_cv, sem_r, sem_g, sem_s):
        cid = lax.axis_index("c")
        tid = lax.axis_index("s")

        def zero_stage():
            z = jnp.zeros((LN,), jnp.float32)

            def zb(r, _):
                for g in range(F // LN):
                    gbuf[r, pl.ds(g * LN, LN)] = z
                return 0
            lax.fori_loop(0, CH, zb, 0)

        def scale_batch(s, alpha):
            base = s * EB

            def grp_body(q, _):
                iv = cvb[s, 1, pl.ds(q * LN, LN)]
                vals16 = lax.bitcast_convert_type(iv, jnp.float32) * alpha
                for i in range(LN):
                    vv = lax.gather(
                        vals16, jnp.full((LN, 1), i, jnp.int32), dnums,
                        slice_sizes=(1,),
                        mode=lax.GatherScatterMode.PROMISE_IN_BOUNDS)
                    e = base + q * LN + i
                    for g in range(F // LN):
                        sl = pl.ds(g * LN, LN)
                        gbuf[e, sl] = gbuf[e, sl] * vv
                return 0
            lax.fori_loop(0, EB // LN, grp_body, 0)

        def cv_start(p, s):
            pltpu.async_copy(cv_hbm.at[cid, tid, p], cvb.at[s], sem_cv.at[s])

        def cv_wait(p, s):
            pltpu.make_async_copy(
                cv_hbm.at[cid, tid, p], cvb.at[s], sem_cv.at[s]).wait()

        def rows_start(p, s):
            pltpu.async_copy(r_hbm.at[tid, p], rbuf.at[s], sem_r.at[s])

        def rows_wait(p, s):
            pltpu.make_async_copy(
                r_hbm.at[tid, p], rbuf.at[s], sem_r.at[s]).wait()

        def gather_start(src_hbm, s):
            pltpu.async_copy(src_hbm.at[cvb.at[s, 0]],
                             gbuf.at[pl.ds(s * EB, EB)], sem_g.at[s])

        def gather_wait(src_hbm, s):
            pltpu.make_async_copy(
                src_hbm.at[cvb.at[s, 0]],
                gbuf.at[pl.ds(s * EB, EB)], sem_g.at[s]).wait()

        def scatter_start(s):
            pltpu.async_copy(gbuf.at[pl.ds(s * EB, EB)],
                             acc_sh.at[rbuf.at[s]], sem_s.at[s], add=True)

        def scatter_wait(s):
            pltpu.make_async_copy(
                gbuf.at[pl.ds(s * EB, EB)],
                acc_sh.at[rbuf.at[s]], sem_s.at[s]).wait()

        def step(src_hbm, prev2_hbm, dst_hbm, alpha):
            if prev2_hbm is None:
                zero_stage()
                for it in range(nit):
                    k = tid + it * NS

                    @pl.when(k < nchk)
                    def _():
                        pltpu.sync_copy(gbuf.at[pl.ds(0, CH)],
                                        acc_sh.at[pl.ds(k * CH, CH)])
            else:
                for it in range(nit):
                    k = tid + it * NS

                    @pl.when(k < nchk)
                    def _():
                        r0 = k * CH
                        g0 = cid * M + r0
                        pltpu.sync_copy(prev2_hbm.at[pl.ds(g0, CH)],
                                        gbuf.at[pl.ds(0, CH)])

                        def neg_body(r, _):
                            for g in range(F // LN):
                                sl = pl.ds(g * LN, LN)
                                gbuf[r, sl] = -gbuf[r, sl]
                            return 0
                        lax.fori_loop(0, CH, neg_body, 0)
                        pltpu.sync_copy(gbuf.at[pl.ds(0, CH)],
                                        acc_sh.at[pl.ds(r0, CH)])
            plsc.subcore_barrier()

            for p0 in range(3):
                cv_start(p0, p0)
            rows_start(0, 0)
            cv_wait(0, 0)
            gather_start(src_hbm, 0)

            def it_body(j, _):
                s = lax.rem(j, 3)
                sn = lax.rem(j + 1, 3)

                @pl.when(j + 1 < NB)
                def _():
                    cv_wait(j + 1, sn)

                    @pl.when(j >= 2)
                    def _():
                        scatter_wait(sn)
                    rows_start(j + 1, sn)
                    gather_start(src_hbm, sn)

                gather_wait(src_hbm, s)
                scale_batch(s, alpha)
                rows_wait(j, s)
                scatter_start(s)

                @pl.when(j + 3 < NB)
                def _():
                    cv_start(j + 3, s)
                return 0
            lax.fori_loop(0, NB, it_body, 0)
            for s in range(3):
                scatter_wait(s)
            plsc.subcore_barrier()

            for it in range(nit):
                k = tid + it * NS

                @pl.when(k < nchk)
                def _():
                    r0 = k * CH
                    g0 = cid * M + r0
                    pltpu.sync_copy(acc_sh.at[pl.ds(r0, CH)],
                                    gbuf.at[pl.ds(0, CH)])
                    pltpu.sync_copy(gbuf.at[pl.ds(0, CH)],
                                    dst_hbm.at[pl.ds(g0, CH)])
            plsc.subcore_barrier()

        step(xin_hbm, None, x1_hbm, 1.0)
        step(x1_hbm, xin_hbm, x2_hbm, 2.0)
        step(x2_hbm, x1_hbm, x3_hbm, 2.0)

    out = jax.ShapeDtypeStruct((NC * M, F), jnp.float32)
    fn = pl.kernel(
        body,
        out_type=(out, out, out),
        mesh=plsc.VectorSubcoreMesh(core_axis_name="c", subcore_axis_name="s"),
        scratch_types=(
            [pltpu.VMEM((3, 2, EB), jnp.int32)]
            + [pltpu.VMEM((3, EB), jnp.int32)]
            + [pltpu.VMEM((3 * EB, F), jnp.float32)]
            + [pltpu.VMEM_SHARED((M, F), jnp.float32)]
            + [pltpu.SemaphoreType.DMA((3,))] * 4
        ),
    )
    return fn(xin, cv, rows)


def _combine_body(x0_ref, x1_ref, x2_ref, x3_ref, w_ref, o_ref):
    acc = jnp.dot(x0_ref[...], w_ref[0], preferred_element_type=jnp.float32)
    acc += jnp.dot(x1_ref[...], w_ref[1], preferred_element_type=jnp.float32)
    acc += jnp.dot(x2_ref[...], w_ref[2], preferred_element_type=jnp.float32)
    acc += jnp.dot(x3_ref[...], w_ref[3], preferred_element_type=jnp.float32)
    o_ref[...] = acc


def _tc_combine(xs, wperm, TM=1000):
    BM, F = xs[0].shape
    Kk, _, Fout = wperm.shape
    xspec = pl.BlockSpec((TM, F), lambda i: (i, 0))
    return pl.pallas_call(
        _combine_body,
        out_shape=jax.ShapeDtypeStruct((BM, Fout), jnp.float32),
        grid=(BM // TM,),
        in_specs=[xspec, xspec, xspec, xspec,
                  pl.BlockSpec((Kk, F, Fout), lambda i: (0, 0, 0))],
        out_specs=pl.BlockSpec((TM, Fout), lambda i: (i, 0)),
    )(*xs, wperm)


def kernel(x, L_indices, L_values, kernel):
    B, M, F = x.shape
    Kk = kernel.shape[0] // F
    Fout = kernel.shape[1]
    NNZ = L_values.shape[0]

    NB = -(-NNZ // (NS * EB))
    pad = NS * NB * EB - NNZ

    row = jnp.concatenate([L_indices[0], jnp.zeros((pad,), jnp.int32)])
    col = jnp.concatenate([L_indices[1], jnp.zeros((pad,), jnp.int32)])
    val = jnp.concatenate([L_values, jnp.zeros((pad,), jnp.float32)])
    vbits = lax.bitcast_convert_type(val, jnp.int32)
    cv = jnp.stack(
        [jnp.stack([(col + c * M).reshape(NS, NB, EB),
                    vbits.reshape(NS, NB, EB)], axis=2) for c in range(B)],
        axis=0)
    rows = row.reshape(NS, NB, EB)

    xin = x.reshape(B * M, F)
    x1, x2, x3 = _sc_chebyshev(M, F, NB, xin, cv, rows)

    wperm = kernel.reshape(F, Kk, Fout).transpose(1, 0, 2)
    out = _tc_combine([xin, x1, x2, x3], wperm)
    return out.reshape(B, M, Fout)


# speedup vs baseline: 1.0034x; 1.0034x over previous
import functools

import jax
import jax.numpy as jnp
from jax import lax
from jax.experimental import pallas as pl
from jax.experimental.pallas import tpu as pltpu
from jax.experimental.pallas import tpu_sc as plsc

NC = 2
NS = 16
LN = 16
EB = 128
CH = 80


def _sc_chebyshev(M, F, NB, xin, cv, rows):
    nchk = M // CH
    nit = -(-nchk // NS)

    dnums = lax.GatherDimensionNumbers(
        offset_dims=(), collapsed_slice_dims=(0,), start_index_map=(0,))

    def body(xin_hbm, cv_hbm, r_hbm, x1_hbm, x2_hbm, x3_hbm,
             cvb, rbuf, gbuf, acc_sh,
             sem_cv, sem_r, sem_g, sem_s):
        cid = lax.axis_index("c")
        tid = lax.axis_index("s")

        def zero_stage():
            z = jnp.zeros((LN,), jnp.float32)

            def zb(r, _):
                for g in range(F // LN):
                    gbuf[r, pl.ds(g * LN, LN)] = z
                return 0
            lax.fori_loop(0, CH, zb, 0)

        def scale_batch(s, alpha):
            base = s * EB

            def grp_body(q, _):
                iv = cvb[s, 1, pl.ds(q * LN, LN)]
                vals16 = lax.bitcast_convert_type(iv, jnp.float32) * alpha
                for i in range(LN):
                    vv = lax.gather(
                        vals16, jnp.full((LN, 1), i, jnp.int32), dnums,
                        slice_sizes=(1,),
                        mode=lax.GatherScatterMode.PROMISE_IN_BOUNDS)
                    e = base + q * LN + i
                    for g in range(F // LN):
                        sl = pl.ds(g * LN, LN)
                        gbuf[e, sl] = gbuf[e, sl] * vv
                return 0
            lax.fori_loop(0, EB // LN, grp_body, 0)

        def cv_start(p, s):
            pltpu.async_copy(cv_hbm.at[cid, tid, p], cvb.at[s], sem_cv.at[s])

        def cv_wait(p, s):
            pltpu.make_async_copy(
                cv_hbm.at[cid, tid, p], cvb.at[s], sem_cv.at[s]).wait()

        def rows_start(p, s):
            pltpu.async_copy(r_hbm.at[tid, p], rbuf.at[s], sem_r.at[s])

        def rows_wait(p, s):
            pltpu.make_async_copy(
                r_hbm.at[tid, p], rbuf.at[s], sem_r.at[s]).wait()

        def gather_start(src_hbm, s):
            pltpu.async_copy(src_hbm.at[cvb.at[s, 0]],
                             gbuf.at[pl.ds(s * EB, EB)], sem_g.at[s])

        def gather_wait(src_hbm, s):
            pltpu.make_async_copy(
                src_hbm.at[cvb.at[s, 0]],
                gbuf.at[pl.ds(s * EB, EB)], sem_g.at[s]).wait()

        def scatter_start(s):
            pltpu.async_copy(gbuf.at[pl.ds(s * EB, EB)],
                             acc_sh.at[rbuf.at[s]], sem_s.at[s], add=True)

        def scatter_wait(s):
            pltpu.make_async_copy(
                gbuf.at[pl.ds(s * EB, EB)],
                acc_sh.at[rbuf.at[s]], sem_s.at[s]).wait()

        def step(src_hbm, prev2_hbm, dst_hbm, alpha):
            if prev2_hbm is None:
                zero_stage()
                for it in range(nit):
                    k = tid + it * NS

                    @pl.when(k < nchk)
                    def _():
                        pltpu.sync_copy(gbuf.at[pl.ds(0, CH)],
                                        acc_sh.at[pl.ds(k * CH, CH)])
            else:
                for it in range(nit):
                    k = tid + it * NS

                    @pl.when(k < nchk)
                    def _():
                        r0 = k * CH
                        g0 = cid * M + r0
                        pltpu.sync_copy(prev2_hbm.at[pl.ds(g0, CH)],
                                        gbuf.at[pl.ds(0, CH)])

                        def neg_body(r, _):
                            for g in range(F // LN):
                                sl = pl.ds(g * LN, LN)
                                gbuf[r, sl] = -gbuf[r, sl]
                            return 0
                        lax.fori_loop(0, CH, neg_body, 0)
                        pltpu.sync_copy(gbuf.at[pl.ds(0, CH)],
                                        acc_sh.at[pl.ds(r0, CH)])
            plsc.subcore_barrier()

            for p0 in range(3):
                cv_start(p0, p0)
            rows_start(0, 0)
            cv_wait(0, 0)
            gather_start(src_hbm, 0)

            def it_body(j, _):
                s = lax.rem(j, 3)
                sn = lax.rem(j + 1, 3)

                @pl.when(j + 1 < NB)
                def _():
                    cv_wait(j + 1, sn)

                    @pl.when(j >= 2)
                    def _():
                        scatter_wait(sn)
                    rows_start(j + 1, sn)
                    gather_start(src_hbm, sn)

                gather_wait(src_hbm, s)
                scale_batch(s, alpha)
                rows_wait(j, s)
                scatter_start(s)

                @pl.when(j + 3 < NB)
                def _():
                    cv_start(j + 3, s)
                return 0
            lax.fori_loop(0, NB, it_body, 0)
            for s in range(3):
                scatter_wait(s)
            plsc.subcore_barrier()

            for it in range(nit):
                k = tid + it * NS

                @pl.when(k < nchk)
                def _():
                    r0 = k * CH
                    g0 = cid * M + r0
                    pltpu.sync_copy(acc_sh.at[pl.ds(r0, CH)],
                                    gbuf.at[pl.ds(0, CH)])
                    pltpu.sync_copy(gbuf.at[pl.ds(0, CH)],
                                    dst_hbm.at[pl.ds(g0, CH)])
            plsc.subcore_barrier()

        step(xin_hbm, None, x1_hbm, 1.0)
        step(x1_hbm, xin_hbm, x2_hbm, 2.0)
        step(x2_hbm, x1_hbm, x3_hbm, 2.0)

    out = jax.ShapeDtypeStruct((NC * M, F), jnp.float32)
    fn = pl.kernel(
        body,
        out_type=(out, out, out),
        mesh=plsc.VectorSubcoreMesh(core_axis_name="c", subcore_axis_name="s"),
        scratch_types=(
            [pltpu.VMEM((3, 2, EB), jnp.int32)]
            + [pltpu.VMEM((3, EB), jnp.int32)]
            + [pltpu.VMEM((3 * EB, F), jnp.float32)]
            + [pltpu.VMEM_SHARED((M, F), jnp.float32)]
            + [pltpu.SemaphoreType.DMA((3,))] * 4
        ),
    )
    return fn(xin, cv, rows)


def _combine_body(x0_ref, x1_ref, x2_ref, x3_ref, w_ref, o_ref):
    acc = jnp.dot(x0_ref[...], w_ref[0], preferred_element_type=jnp.float32)
    acc += jnp.dot(x1_ref[...], w_ref[1], preferred_element_type=jnp.float32)
    acc += jnp.dot(x2_ref[...], w_ref[2], preferred_element_type=jnp.float32)
    acc += jnp.dot(x3_ref[...], w_ref[3], preferred_element_type=jnp.float32)
    o_ref[...] = acc


def _tc_combine(xs, wperm, TM=1000):
    BM, F = xs[0].shape
    Kk, _, Fout = wperm.shape
    xspec = pl.BlockSpec((TM, F), lambda i: (i, 0))
    return pl.pallas_call(
        _combine_body,
        out_shape=jax.ShapeDtypeStruct((BM, Fout), jnp.float32),
        grid=(BM // TM,),
        in_specs=[xspec, xspec, xspec, xspec,
                  pl.BlockSpec((Kk, F, Fout), lambda i: (0, 0, 0))],
        out_specs=pl.BlockSpec((TM, Fout), lambda i: (i, 0)),
    )(*xs, wperm)


def kernel(x, L_indices, L_values, kernel):
    B, M, F = x.shape
    Kk = kernel.shape[0] // F
    Fout = kernel.shape[1]
    NNZ = L_values.shape[0]

    NB = -(-NNZ // (NS * EB))
    pad = NS * NB * EB - NNZ

    row = jnp.concatenate([L_indices[0], jnp.zeros((pad,), jnp.int32)])
    col = jnp.concatenate([L_indices[1], jnp.zeros((pad,), jnp.int32)])
    val = jnp.concatenate([L_values, jnp.zeros((pad,), jnp.float32)])
    vbits = lax.bitcast_convert_type(val, jnp.int32)
    cv = jnp.stack(
        [jnp.stack([(col + c * M).reshape(NS, NB, EB),
                    vbits.reshape(NS, NB, EB)], axis=2) for c in range(B)],
        axis=0)
    rows = row.reshape(NS, NB, EB)

    xin = x.reshape(B * M, F)
    x1, x2, x3 = _sc_chebyshev(M, F, NB, xin, cv, rows)

    wperm = kernel.reshape(F, Kk, Fout).transpose(1, 0, 2)
    out = _tc_combine([xin, x1, x2, x3], wperm)
    return out.reshape(B, M, Fout)


# speedup vs baseline: 1.0801x; 1.0765x over previous
import functools

import jax
import jax.numpy as jnp
from jax import lax
from jax.experimental import pallas as pl
from jax.experimental.pallas import tpu as pltpu
from jax.experimental.pallas import tpu_sc as plsc

NC = 2
NS = 16
LN = 16
EB = 128
CH = 80


def _sc_chebyshev(M, F, NB, xin, cv, rows):
    nchk = M // CH
    nit = -(-nchk // NS)

    dnums = lax.GatherDimensionNumbers(
        offset_dims=(), collapsed_slice_dims=(0,), start_index_map=(0,))

    def body(xin_hbm, cv_hbm, r_hbm, x1_hbm, x2_hbm, x3_hbm,
             cvb, rbuf, gbuf, acc_sh,
             sem_cv, sem_r, sem_g, sem_s):
        cid = lax.axis_index("c")
        tid = lax.axis_index("s")
        coff = jnp.full((LN,), cid * M, dtype=jnp.int32)

        def zero_stage():
            z = jnp.zeros((LN,), jnp.float32)

            def zb(r, _):
                for g in range(F // LN):
                    gbuf[r, pl.ds(g * LN, LN)] = z
                return 0
            lax.fori_loop(0, CH, zb, 0)

        def scale_batch(s, alpha):
            base = s * EB

            def grp_body(q, _):
                iv = cvb[s, 1, pl.ds(q * LN, LN)]
                vals16 = lax.bitcast_convert_type(iv, jnp.float32) * alpha
                for i in range(LN):
                    vv = lax.gather(
                        vals16, jnp.full((LN, 1), i, jnp.int32), dnums,
                        slice_sizes=(1,),
                        mode=lax.GatherScatterMode.PROMISE_IN_BOUNDS)
                    e = base + q * LN + i
                    for g in range(F // LN):
                        sl = pl.ds(g * LN, LN)
                        gbuf[e, sl] = gbuf[e, sl] * vv
                return 0
            lax.fori_loop(0, EB // LN, grp_body, 0)

        def cv_start(p, s):
            pltpu.async_copy(cv_hbm.at[tid, p], cvb.at[s], sem_cv.at[s])

        def cv_wait(p, s):
            pltpu.make_async_copy(
                cv_hbm.at[tid, p], cvb.at[s], sem_cv.at[s]).wait()

        def rows_start(p, s):
            pltpu.async_copy(r_hbm.at[tid, p], rbuf.at[s], sem_r.at[s])

        def rows_wait(p, s):
            pltpu.make_async_copy(
                r_hbm.at[tid, p], rbuf.at[s], sem_r.at[s]).wait()

        def gather_start(src_hbm, s):
            for q in range(EB // LN):
                sl = pl.ds(q * LN, LN)
                cvb[s, 0, sl] = cvb[s, 0, sl] + coff
            pltpu.async_copy(src_hbm.at[cvb.at[s, 0]],
                             gbuf.at[pl.ds(s * EB, EB)], sem_g.at[s])

        def gather_wait(src_hbm, s):
            pltpu.make_async_copy(
                src_hbm.at[cvb.at[s, 0]],
                gbuf.at[pl.ds(s * EB, EB)], sem_g.at[s]).wait()

        def scatter_start(s):
            pltpu.async_copy(gbuf.at[pl.ds(s * EB, EB)],
                             acc_sh.at[rbuf.at[s]], sem_s.at[s], add=True)

        def scatter_wait(s):
            pltpu.make_async_copy(
                gbuf.at[pl.ds(s * EB, EB)],
                acc_sh.at[rbuf.at[s]], sem_s.at[s]).wait()

        def step(src_hbm, prev2_hbm, dst_hbm, alpha):
            if prev2_hbm is None:
                zero_stage()
                for it in range(nit):
                    k = tid + it * NS

                    @pl.when(k < nchk)
                    def _():
                        pltpu.sync_copy(gbuf.at[pl.ds(0, CH)],
                                        acc_sh.at[pl.ds(k * CH, CH)])
            else:
                for it in range(nit):
                    k = tid + it * NS

                    @pl.when(k < nchk)
                    def _():
                        r0 = k * CH
                        g0 = cid * M + r0
                        pltpu.sync_copy(prev2_hbm.at[pl.ds(g0, CH)],
                                        gbuf.at[pl.ds(0, CH)])

                        def neg_body(r, _):
                            for g in range(F // LN):
                                sl = pl.ds(g * LN, LN)
                                gbuf[r, sl] = -gbuf[r, sl]
                            return 0
                        lax.fori_loop(0, CH, neg_body, 0)
                        pltpu.sync_copy(gbuf.at[pl.ds(0, CH)],
                                        acc_sh.at[pl.ds(r0, CH)])
            plsc.subcore_barrier()

            for p0 in range(3):
                cv_start(p0, p0)
            rows_start(0, 0)
            cv_wait(0, 0)
            gather_start(src_hbm, 0)

            def it_body(j, _):
                s = lax.rem(j, 3)
                sn = lax.rem(j + 1, 3)

                @pl.when(j + 1 < NB)
                def _():
                    cv_wait(j + 1, sn)

                    @pl.when(j >= 2)
                    def _():
                        scatter_wait(sn)
                    rows_start(j + 1, sn)
                    gather_start(src_hbm, sn)

                gather_wait(src_hbm, s)
                scale_batch(s, alpha)
                rows_wait(j, s)
                scatter_start(s)

                @pl.when(j + 3 < NB)
                def _():
                    cv_start(j + 3, s)
                return 0
            lax.fori_loop(0, NB, it_body, 0)
            for s in range(3):
                scatter_wait(s)
            plsc.subcore_barrier()

            for it in range(nit):
                k = tid + it * NS

                @pl.when(k < nchk)
                def _():
                    r0 = k * CH
                    g0 = cid * M + r0
                    pltpu.sync_copy(acc_sh.at[pl.ds(r0, CH)],
                                    gbuf.at[pl.ds(0, CH)])
                    pltpu.sync_copy(gbuf.at[pl.ds(0, CH)],
                                    dst_hbm.at[pl.ds(g0, CH)])
            plsc.subcore_barrier()

        step(xin_hbm, None, x1_hbm, 1.0)
        step(x1_hbm, xin_hbm, x2_hbm, 2.0)
        step(x2_hbm, x1_hbm, x3_hbm, 2.0)

    out = jax.ShapeDtypeStruct((NC * M, F), jnp.float32)
    fn = pl.kernel(
        body,
        out_type=(out, out, out),
        mesh=plsc.VectorSubcoreMesh(core_axis_name="c", subcore_axis_name="s"),
        scratch_types=(
            [pltpu.VMEM((3, 2, EB), jnp.int32)]
            + [pltpu.VMEM((3, EB), jnp.int32)]
            + [pltpu.VMEM((3 * EB, F), jnp.float32)]
            + [pltpu.VMEM_SHARED((M, F), jnp.float32)]
            + [pltpu.SemaphoreType.DMA((3,))] * 4
        ),
    )
    return fn(xin, cv, rows)


def _combine_body(x0_ref, x1_ref, x2_ref, x3_ref, w_ref, o_ref):
    acc = jnp.dot(x0_ref[...], w_ref[0], preferred_element_type=jnp.float32)
    acc += jnp.dot(x1_ref[...], w_ref[1], preferred_element_type=jnp.float32)
    acc += jnp.dot(x2_ref[...], w_ref[2], preferred_element_type=jnp.float32)
    acc += jnp.dot(x3_ref[...], w_ref[3], preferred_element_type=jnp.float32)
    o_ref[...] = acc


def _tc_combine(xs, wperm, TM=1000):
    BM, F = xs[0].shape
    Kk, _, Fout = wperm.shape
    xspec = pl.BlockSpec((TM, F), lambda i: (i, 0))
    return pl.pallas_call(
        _combine_body,
        out_shape=jax.ShapeDtypeStruct((BM, Fout), jnp.float32),
        grid=(BM // TM,),
        in_specs=[xspec, xspec, xspec, xspec,
                  pl.BlockSpec((Kk, F, Fout), lambda i: (0, 0, 0))],
        out_specs=pl.BlockSpec((TM, Fout), lambda i: (i, 0)),
    )(*xs, wperm)


def kernel(x, L_indices, L_values, kernel):
    B, M, F = x.shape
    Kk = kernel.shape[0] // F
    Fout = kernel.shape[1]
    NNZ = L_values.shape[0]

    NB = -(-NNZ // (NS * EB))
    pad = NS * NB * EB - NNZ

    row = jnp.concatenate([L_indices[0], jnp.zeros((pad,), jnp.int32)])
    col = jnp.concatenate([L_indices[1], jnp.zeros((pad,), jnp.int32)])
    val = jnp.concatenate([L_values, jnp.zeros((pad,), jnp.float32)])
    vbits = lax.bitcast_convert_type(val, jnp.int32)
    cv = jnp.stack([a.reshape(NS, NB, EB) for a in (col, vbits)], axis=2)
    rows = row.reshape(NS, NB, EB)

    xin = x.reshape(B * M, F)
    x1, x2, x3 = _sc_chebyshev(M, F, NB, xin, cv, rows)

    wperm = kernel.reshape(F, Kk, Fout).transpose(1, 0, 2)
    out = _tc_combine([xin, x1, x2, x3], wperm)
    return out.reshape(B, M, Fout)
